# baseline (speedup 1.0000x reference)
import jax
import jax.numpy as jnp
from jax import lax
from jax.experimental import pallas as pl
from jax.experimental.pallas import tpu as pltpu
from jax.experimental.pallas import tpu_sc as plsc

L = 16
NC = 2
NS = 16
NW = NC * NS
ROWS = 128
COLS = 32768
SC_ROWS = 64
TC_ROWS = ROWS - SC_ROWS
TC_BLK = 8
ROWS_PER_W = SC_ROWS // NW
STEPS = COLS // L
NBLK = 32
BSTEPS = STEPS // NBLK
UNROLL = 8
NACC = 4
BIG = 1 << 30


def _merge(ma, sa, mb, sb, lane):
    ca = sa * L + lane
    cb = sb * L + lane
    c = (mb > ma) | ((mb == ma) & (cb < ca))
    return jnp.where(c, mb, ma), jnp.where(c, sb, sa)


def _merge_accs(ms, ss, lane):
    m01, s01 = _merge(ms[0], ss[0], ms[1], ss[1], lane)
    m23, s23 = _merge(ms[2], ss[2], ms[3], ss[3], lane)
    return _merge(m01, s01, m23, s23, lane)


def _summarize_block(ref, b):
    def inner(j, carry):
        ms = list(carry[:NACC])
        ss = list(carry[NACC:])
        for u in range(UNROLL):
            g = b * BSTEPS + j * UNROLL + u
            v = ref[pl.ds(g * L, L)]
            a = u % NACC
            c = v > ms[a]
            ms[a] = jnp.maximum(ms[a], v)
            ss[a] = jnp.where(c, g, ss[a])
        return tuple(ms) + tuple(ss)

    m0 = jnp.full((L,), -jnp.inf, dtype=jnp.float32)
    s0 = jnp.zeros((L,), dtype=jnp.int32)
    carry = lax.fori_loop(0, BSTEPS // UNROLL, inner, (m0,) * NACC + (s0,) * NACC)
    lane = lax.iota(jnp.int32, L)
    return _merge_accs(list(carry[:NACC]), list(carry[NACC:]), lane)


def _scan_summaries(bmax_ref, bstep_ref):
    def inner(j, carry):
        ms = list(carry[:NACC])
        ss = list(carry[NACC:])
        for u in range(NACC):
            b = j * NACC + u
            mv = bmax_ref[pl.ds(b * L, L)]
            sv = bstep_ref[pl.ds(b * L, L)]
            c = mv > ms[u]
            ms[u] = jnp.maximum(ms[u], mv)
            ss[u] = jnp.where(c, sv, ss[u])
        return tuple(ms) + tuple(ss)

    m0 = jnp.full((L,), -jnp.inf, dtype=jnp.float32)
    s0 = jnp.zeros((L,), dtype=jnp.int32)
    carry = lax.fori_loop(0, NBLK // NACC, inner, (m0,) * NACC + (s0,) * NACC)
    lane = lax.iota(jnp.int32, L)
    return _merge_accs(list(carry[:NACC]), list(carry[NACC:]), lane)


def _fold(v, scratch, op):
    lane = lax.iota(jnp.int32, L)
    for base in (1, 4):
        scratch[...] = v
        for k in (base, 2 * base, 3 * base):
            v = op(v, plsc.load_gather(scratch, [lane ^ k]))
    return v


def _body(x_hbm, out_hbm, buf0, buf1, bmax, bstep, outv, scr_f, scr_i,
          sem0, sem1):
    wid = lax.axis_index("s") * NC + lax.axis_index("c")
    base = wid * ROWS_PER_W
    bufs = (buf0, buf1)
    sems = (sem0, sem1)
    lane = lax.iota(jnp.int32, L)
    neg = jnp.full((L,), -jnp.inf, dtype=jnp.float32)

    cps = [pltpu.async_copy(x_hbm.at[base], bufs[0], sems[0]), None]
    for rr in range(ROWS_PER_W):
        if rr + 1 < ROWS_PER_W:
            nb = (rr + 1) % 2
            cps[nb] = pltpu.async_copy(x_hbm.at[base + (rr + 1)], bufs[nb], sems[nb])
        cps[rr % 2].wait()
        ref = bufs[rr % 2]

        def mainb(b, z, ref=ref):
            m, s = _summarize_block(ref, b)
            bmax[pl.ds(b * L, L)] = m
            bstep[pl.ds(b * L, L)] = s
            return z

        lax.fori_loop(0, NBLK, mainb, 0)

        def extract(p, res, ref=ref):
            m, s = _scan_summaries(bmax, bstep)
            col = s * L + lane
            mxv = _fold(m, scr_f, jnp.maximum)
            cand = jnp.where(m == mxv, col, BIG)
            iv = _fold(cand, scr_i, jnp.minimum)
            i1s = iv[0]
            plsc.store_scatter(ref, [iv], neg, mask=lane == 0)
            b1 = lax.shift_right_logical(i1s, 10)
            m2, s2 = _summarize_block(ref, b1)
            bmax[pl.ds(b1 * L, L)] = m2
            bstep[pl.ds(b1 * L, L)] = s2
            return jnp.where(lane == p, iv, res)

        res = lax.fori_loop(0, 3, extract, jnp.zeros((L,), dtype=jnp.int32))
        outv[rr] = res
    pltpu.sync_copy(outv, out_hbm.at[pl.ds(base, ROWS_PER_W)])


TC_ACC = 4
TC_CHUNKS = COLS // 128
NEG = float("-inf")


def _ins3(t, v, sv, tie):
    m1, s1, m2, s2, m3, s3 = t
    if tie:
        g1 = (v > m1) | ((v == m1) & (sv < s1))
        g2 = (v > m2) | ((v == m2) & (sv < s2))
        g3 = (v > m3) | ((v == m3) & (sv < s3))
    else:
        g1 = v > m1
        g2 = v > m2
        g3 = v > m3
    nm3 = jnp.where(g2, m2, jnp.where(g3, v, m3))
    ns3 = jnp.where(g2, s2, jnp.where(g3, sv, s3))
    nm2 = jnp.where(g1, m1, jnp.where(g2, v, m2))
    ns2 = jnp.where(g1, s1, jnp.where(g2, sv, s2))
    nm1 = jnp.where(g1, v, m1)
    ns1 = jnp.where(g1, sv, s1)
    return [nm1, ns1, nm2, ns2, nm3, ns3]


def _tc_body(x_ref, o_ref):
    lane = lax.broadcasted_iota(jnp.int32, (TC_BLK, 128), 1)
    lane16 = lax.broadcasted_iota(jnp.int32, (TC_BLK, 3), 1)
    mneg = jnp.full((TC_BLK, 128), NEG, dtype=jnp.float32)
    zero = jnp.zeros((TC_BLK, 128), dtype=jnp.int32)
    accs = [[mneg, zero, mneg, zero, mneg, zero] for _ in range(TC_ACC)]
    for j in range(TC_CHUNKS):
        ch = x_ref[:, j * 128:(j + 1) * 128]
        a = j % TC_ACC
        accs[a] = _ins3(accs[a], ch, j, tie=False)
    while len(accs) > 1:
        nxt = []
        for t in range(0, len(accs), 2):
            dst, src = accs[t], accs[t + 1]
            for q in (0, 2, 4):
                dst = _ins3(dst, src[q], src[q + 1], tie=True)
            nxt.append(dst)
        accs = nxt
    m1, s1, m2, s2, m3, s3 = accs[0]
    col1 = s1 * 128 + lane
    col2 = s2 * 128 + lane
    col3 = s3 * 128 + lane
    outs = []
    for _ in range(3):
        mx = jnp.max(m1, axis=1, keepdims=True)
        cand = jnp.where(m1 == mx, col1, BIG)
        i = jnp.min(cand, axis=1, keepdims=True)
        outs.append(i)
        eq = col1 == i
        m1 = jnp.where(eq, m2, m1)
        col1 = jnp.where(eq, col2, col1)
        m2 = jnp.where(eq, m3, m2)
        col2 = jnp.where(eq, col3, col2)
        m3 = jnp.where(eq, NEG, m3)
    o_ref[...] = jnp.where(
        lane16 == 0, outs[0], jnp.where(lane16 == 1, outs[1], outs[2])
    )


def _topk3_tc(x):
    return pl.pallas_call(
        _tc_body,
        grid=(TC_ROWS // TC_BLK,),
        in_specs=[pl.BlockSpec((TC_BLK, COLS), lambda i: (i + SC_ROWS // TC_BLK, 0))],
        out_specs=pl.BlockSpec((TC_BLK, 3), lambda i: (i, 0)),
        out_shape=jax.ShapeDtypeStruct((TC_ROWS, 3), jnp.int32),
    )(x)


@jax.jit
def _topk3(x):
    mesh = plsc.VectorSubcoreMesh(core_axis_name="c", subcore_axis_name="s")
    run = pl.kernel(
        _body,
        out_type=jax.ShapeDtypeStruct((SC_ROWS, L), jnp.int32),
        mesh=mesh,
        compiler_params=pltpu.CompilerParams(needs_layout_passes=False),
        scratch_types=[
            pltpu.VMEM((COLS,), jnp.float32),
            pltpu.VMEM((COLS,), jnp.float32),
            pltpu.VMEM((NBLK * L,), jnp.float32),
            pltpu.VMEM((NBLK * L,), jnp.int32),
            pltpu.VMEM((ROWS_PER_W, L), jnp.int32),
            pltpu.VMEM((L,), jnp.float32),
            pltpu.VMEM((L,), jnp.int32),
            pltpu.SemaphoreType.DMA,
            pltpu.SemaphoreType.DMA,
        ],
    )
    sc_out = run(x)
    tc_out = _topk3_tc(x)
    return jnp.concatenate([sc_out[:, :3], tc_out], axis=0)


def kernel(x):
    return _topk3(x)
